# baseline (speedup 1.0000x reference)
import functools

import jax
import jax.numpy as jnp
from jax import lax
from jax.experimental import pallas as pl
from jax.experimental.pallas import tpu as pltpu
from jax.experimental.pallas import tpu_sc as plsc

VOCAB = 49408
MAX_LEN = 77
DIM = 768
BATCH = 1024
SEQ = 77
N = BATCH * SEQ
NW = 32
PER_W = N // NW
CHUNK = 16
NCH = PER_W // CHUNK
NBUF = 4
LANES = 16
UNROLL = 2
SEG = DIM // LANES
KTOT = (NCH + 1 + NBUF - 1) // NBUF


_mesh = plsc.VectorSubcoreMesh(core_axis_name="c", subcore_axis_name="s")


@functools.partial(
    pl.kernel,
    mesh=_mesh,
    out_type=jax.ShapeDtypeStruct((N, DIM), jnp.float32),
    compiler_params=pltpu.CompilerParams(needs_layout_passes=False),
    scratch_types=[
        pltpu.VMEM((PER_W,), jnp.int32),
        pltpu.VMEM((PER_W,), jnp.int32),
        pltpu.VMEM((MAX_LEN * DIM,), jnp.float32),
        pltpu.VMEM((NBUF, CHUNK, DIM), jnp.float32),
        pltpu.SemaphoreType.DMA,
        pltpu.SemaphoreType.DMA,
        pltpu.SemaphoreType.DMA,
        pltpu.SemaphoreType.DMA,
        pltpu.SemaphoreType.DMA,
        pltpu.SemaphoreType.DMA,
        pltpu.SemaphoreType.DMA,
        pltpu.SemaphoreType.DMA,
    ],
)
def _emb(tok_idx, pos_idx, tok_tab, pos_tab, out, idx_t, idx_p, pos_v,
         tok_buf, sg0, sg1, sg2, sg3, sw0, sw1, sw2, sw3):
    wid = lax.axis_index("s") * 2 + lax.axis_index("c")
    sem_g = (sg0, sg1, sg2, sg3)
    sem_w = (sw0, sw1, sw2, sw3)

    pltpu.sync_copy(tok_idx.at[pl.ds(wid * PER_W, PER_W)], idx_t)
    pltpu.sync_copy(pos_idx.at[pl.ds(wid * PER_W, PER_W)], idx_p)
    pltpu.sync_copy(pos_tab, pos_v)

    def start_gather(i, b):
        pltpu.async_copy(tok_tab.at[idx_t.at[pl.ds(i * CHUNK, CHUNK)]],
                         tok_buf.at[b], sem_g[b])

    def wait_gather(i, b):
        pltpu.make_async_copy(tok_tab.at[idx_t.at[pl.ds(i * CHUNK, CHUNK)]],
                              tok_buf.at[b], sem_g[b]).wait()

    def out_rows(i):
        return out.at[pl.ds((wid * NCH + i) * CHUNK, CHUNK)]

    def start_wb(i, b):
        pltpu.async_copy(tok_buf.at[b], out_rows(i), sem_w[b])

    def wait_wb(i, b):
        pltpu.make_async_copy(tok_buf.at[b], out_rows(i), sem_w[b]).wait()

    lane = lax.iota(jnp.int32, LANES)

    def compute(i, b):
        p_vec = idx_p[pl.ds(i * CHUNK, CHUNK)]

        @plsc.parallel_loop(0, CHUNK, step=1, unroll=UNROLL)
        def _(r):
            pr = lax.gather(
                p_vec, jnp.full((LANES, 1), r, dtype=jnp.int32),
                lax.GatherDimensionNumbers(offset_dims=(),
                                           collapsed_slice_dims=(0,),
                                           start_index_map=(0,)),
                slice_sizes=(1,),
                mode=lax.GatherScatterMode.PROMISE_IN_BOUNDS)
            base = pr * DIM + lane
            for s in range(SEG):
                pv = plsc.load_gather(
                    pos_v.at[pl.ds(s * LANES, MAX_LEN * DIM - s * LANES)],
                    [base])
                plsc.addupdate(tok_buf.at[b].at[r, pl.ds(s * LANES, LANES)],
                               pv)

    for _b in range(NBUF - 1):
        start_gather(_b, _b)

    def outer(k, carry):
        for j in range(NBUF):
            i = NBUF * k + j
            bg = (j + NBUF - 1) % NBUF

            @pl.when(jnp.logical_and(i >= 1, i - 1 < NCH))
            def _():
                wait_wb(i - 1, bg)

            @pl.when(i + NBUF - 1 < NCH)
            def _():
                start_gather(i + NBUF - 1, bg)

            @pl.when(i < NCH)
            def _():
                wait_gather(i, j)
                compute(i, j)
                start_wb(i, j)
        return carry

    lax.fori_loop(0, KTOT, outer, 0)


def kernel(tokens, positions, token_table, pos_table):
    tok = tokens.reshape(N).astype(jnp.int32)
    pos = positions.reshape(N).astype(jnp.int32)
    out = _emb(tok, pos, token_table, pos_table.reshape(MAX_LEN * DIM))
    return out.reshape(BATCH, SEQ, DIM)


# speedup vs baseline: 1.1171x; 1.1171x over previous
import functools

import jax
import jax.numpy as jnp
from jax import lax
from jax.experimental import pallas as pl
from jax.experimental.pallas import tpu as pltpu
from jax.experimental.pallas import tpu_sc as plsc

VOCAB = 49408
MAX_LEN = 77
DIM = 768
BATCH = 1024
SEQ = 77
N = BATCH * SEQ
NW = 32
PER_W = N // NW
CHUNK = 16
NCH = PER_W // CHUNK
NBUF = 3
LANES = 16
UNROLL = 1
SEG = DIM // LANES
KTOT = (NCH + 1 + NBUF - 1) // NBUF


_mesh = plsc.VectorSubcoreMesh(core_axis_name="c", subcore_axis_name="s")


@functools.partial(
    pl.kernel,
    mesh=_mesh,
    out_type=jax.ShapeDtypeStruct((N, DIM), jnp.float32),
    compiler_params=pltpu.CompilerParams(needs_layout_passes=False),
    scratch_types=[
        pltpu.VMEM((PER_W,), jnp.int32),
        pltpu.VMEM((PER_W,), jnp.int32),
        pltpu.VMEM((MAX_LEN * DIM,), jnp.float32),
        pltpu.VMEM((NBUF, CHUNK, DIM), jnp.float32),
        pltpu.SemaphoreType.DMA,
        pltpu.SemaphoreType.DMA,
        pltpu.SemaphoreType.DMA,
        pltpu.SemaphoreType.DMA,
        pltpu.SemaphoreType.DMA,
        pltpu.SemaphoreType.DMA,
    ],
)
def _emb(tok_idx, pos_idx, tok_tab, pos_tab, out, idx_t, idx_p, pos_v,
         tok_buf, sg0, sg1, sg2, sw0, sw1, sw2):
    wid = lax.axis_index("s") * 2 + lax.axis_index("c")
    sem_g = (sg0, sg1, sg2)
    sem_w = (sw0, sw1, sw2)

    pltpu.sync_copy(tok_idx.at[pl.ds(wid * PER_W, PER_W)], idx_t)
    pltpu.sync_copy(pos_idx.at[pl.ds(wid * PER_W, PER_W)], idx_p)
    pltpu.sync_copy(pos_tab, pos_v)

    def start_gather(i, b):
        pltpu.async_copy(tok_tab.at[idx_t.at[pl.ds(i * CHUNK, CHUNK)]],
                         tok_buf.at[b], sem_g[b])

    def wait_gather(i, b):
        pltpu.make_async_copy(tok_tab.at[idx_t.at[pl.ds(i * CHUNK, CHUNK)]],
                              tok_buf.at[b], sem_g[b]).wait()

    def out_rows(i):
        return out.at[pl.ds((wid * NCH + i) * CHUNK, CHUNK)]

    def start_wb(i, b):
        pltpu.async_copy(tok_buf.at[b], out_rows(i), sem_w[b])

    def wait_wb(i, b):
        pltpu.make_async_copy(tok_buf.at[b], out_rows(i), sem_w[b]).wait()

    lane = lax.iota(jnp.int32, LANES)

    def compute(i, b):
        p_vec = idx_p[pl.ds(i * CHUNK, CHUNK)]

        @plsc.parallel_loop(0, CHUNK, step=1, unroll=UNROLL)
        def _(r):
            pr = lax.gather(
                p_vec, jnp.full((LANES, 1), r, dtype=jnp.int32),
                lax.GatherDimensionNumbers(offset_dims=(),
                                           collapsed_slice_dims=(0,),
                                           start_index_map=(0,)),
                slice_sizes=(1,),
                mode=lax.GatherScatterMode.PROMISE_IN_BOUNDS)
            base = pr * DIM + lane
            for s in range(SEG):
                pv = plsc.load_gather(
                    pos_v.at[pl.ds(s * LANES, MAX_LEN * DIM - s * LANES)],
                    [base])
                plsc.addupdate(tok_buf.at[b].at[r, pl.ds(s * LANES, LANES)],
                               pv)

    for _b in range(NBUF - 1):
        start_gather(_b, _b)

    def outer(k, carry):
        for j in range(NBUF):
            i = NBUF * k + j
            bg = (j + NBUF - 1) % NBUF

            @pl.when(jnp.logical_and(i >= 1, i - 1 < NCH))
            def _():
                wait_wb(i - 1, bg)

            @pl.when(i + NBUF - 1 < NCH)
            def _():
                start_gather(i + NBUF - 1, bg)

            @pl.when(i < NCH)
            def _():
                wait_gather(i, j)
                compute(i, j)
                start_wb(i, j)
        return carry

    lax.fori_loop(0, KTOT, outer, 0)


def kernel(tokens, positions, token_table, pos_table):
    tok = tokens.reshape(N).astype(jnp.int32)
    pos = positions.reshape(N).astype(jnp.int32)
    out = _emb(tok, pos, token_table, pos_table.reshape(MAX_LEN * DIM))
    return out.reshape(BATCH, SEQ, DIM)
